# baseline (speedup 1.0000x reference)
import functools

import jax
import jax.numpy as jnp
from jax import lax
from jax.experimental import pallas as pl
from jax.experimental.pallas import tpu as pltpu
from jax.experimental.pallas import tpu_sc as plsc

DIM = 128
B = 4096 * 50
NC, NS = 2, 16
NW = NC * NS
BPW = B // NW
CHUNK = 128
NCHUNKS = BPW // CHUNK
NBUF = 5
NROUNDS = NCHUNKS // NBUF - 1

_mesh = plsc.VectorSubcoreMesh(core_axis_name="c", subcore_axis_name="s")


@functools.partial(
    pl.kernel,
    mesh=_mesh,
    out_type=jax.ShapeDtypeStruct((B, DIM), jnp.float32),
    scratch_types=[
        pltpu.VMEM((NCHUNKS, CHUNK), jnp.int32),
        pltpu.VMEM((NBUF, CHUNK, DIM), jnp.float32),
        [pltpu.SemaphoreType.DMA for _ in range(NBUF)],
        [pltpu.SemaphoreType.DMA for _ in range(NBUF)],
    ],
)
def _embed_gather(idx_hbm, table_hbm, out_hbm, idx_v, rows_v, gsems, ssems):
    wid = lax.axis_index("s") * NC + lax.axis_index("c")
    base = wid * BPW
    pltpu.sync_copy(idx_hbm.at[wid], idx_v)

    for b in range(NBUF):
        pltpu.async_copy(table_hbm.at[idx_v.at[b]], rows_v.at[b], gsems[b])

    def body(g, carry):
        c0 = g * NBUF
        for b in range(NBUF):
            c = c0 + b
            pltpu.make_async_copy(
                table_hbm.at[idx_v.at[c]], rows_v.at[b], gsems[b]).wait()
            pltpu.async_copy(rows_v.at[b],
                             out_hbm.at[pl.ds(base + c * CHUNK, CHUNK)],
                             ssems[b])
        for b in range(NBUF):
            c = c0 + b
            pltpu.make_async_copy(
                rows_v.at[b], out_hbm.at[pl.ds(base + c * CHUNK, CHUNK)],
                ssems[b]).wait()
            pltpu.async_copy(table_hbm.at[idx_v.at[c + NBUF]],
                             rows_v.at[b], gsems[b])
        return carry

    lax.fori_loop(0, NROUNDS, body, 0)

    for b in range(NBUF):
        c = NROUNDS * NBUF + b
        pltpu.make_async_copy(
            table_hbm.at[idx_v.at[c]], rows_v.at[b], gsems[b]).wait()
        pltpu.async_copy(rows_v.at[b],
                         out_hbm.at[pl.ds(base + c * CHUNK, CHUNK)], ssems[b])
    for b in range(NBUF):
        c = NROUNDS * NBUF + b
        pltpu.make_async_copy(
            rows_v.at[b], out_hbm.at[pl.ds(base + c * CHUNK, CHUNK)],
            ssems[b]).wait()


def kernel(x, weight):
    idx = x.reshape(NW, NCHUNKS, CHUNK).astype(jnp.int32)
    out = _embed_gather(idx, weight)
    return out.reshape(x.shape + (weight.shape[1],))


# speedup vs baseline: 1.7775x; 1.7775x over previous
import functools

import jax
import jax.numpy as jnp
from jax import lax
from jax.experimental import pallas as pl
from jax.experimental.pallas import tpu as pltpu
from jax.experimental.pallas import tpu_sc as plsc

N_I = 4096
N_J = 50
DIM = 128
NC, NS = 2, 16
NW = NC * NS
IPW = N_I // NW
KI = 4
NSUPER = IPW // KI
NBUF = 4
NROUNDS = NSUPER // NBUF - 1

_mesh = plsc.VectorSubcoreMesh(core_axis_name="c", subcore_axis_name="s")


@functools.partial(
    pl.kernel,
    mesh=_mesh,
    out_type=jax.ShapeDtypeStruct((N_I, N_J, DIM), jnp.float32),
    scratch_types=[
        pltpu.VMEM((IPW, N_J), jnp.int32),
        pltpu.VMEM((NBUF, KI, N_J, DIM), jnp.float32),
        [pltpu.SemaphoreType.DMA for _ in range(NBUF)],
        [pltpu.SemaphoreType.DMA for _ in range(NBUF)],
    ],
)
def _embed_gather(idx_hbm, table_hbm, out_hbm, idx_v, rows_v, gsems, ssems):
    wid = lax.axis_index("s") * NC + lax.axis_index("c")
    ibase = wid * IPW
    pltpu.sync_copy(idx_hbm.at[pl.ds(ibase, IPW)], idx_v)

    def start_gathers(c, b):
        for k in range(KI):
            pltpu.async_copy(table_hbm.at[idx_v.at[c * KI + k]],
                             rows_v.at[b, k], gsems[b])

    def wait_gathers(c, b):
        for k in range(KI):
            pltpu.make_async_copy(table_hbm.at[idx_v.at[c * KI + k]],
                                  rows_v.at[b, k], gsems[b]).wait()

    def store(c, b):
        return pltpu.make_async_copy(
            rows_v.at[b], out_hbm.at[pl.ds(ibase + c * KI, KI)], ssems[b])

    for b in range(NBUF):
        start_gathers(b, b)

    def body(g, carry):
        c0 = g * NBUF
        for b in range(NBUF):
            wait_gathers(c0 + b, b)
            store(c0 + b, b).start()
        for b in range(NBUF):
            store(c0 + b, b).wait()
            start_gathers(c0 + b + NBUF, b)
        return carry

    lax.fori_loop(0, NROUNDS, body, 0)

    c0 = NROUNDS * NBUF
    for b in range(NBUF):
        wait_gathers(c0 + b, b)
        store(c0 + b, b).start()
    for b in range(NBUF):
        store(c0 + b, b).wait()


def kernel(x, weight):
    return _embed_gather(x.astype(jnp.int32), weight)


# speedup vs baseline: 3.0989x; 1.7434x over previous
import functools

import jax
import jax.numpy as jnp
from jax import lax
from jax.experimental import pallas as pl
from jax.experimental.pallas import tpu as pltpu
from jax.experimental.pallas import tpu_sc as plsc

N_I = 4096
N_J = 50
DIM = 128
NC, NS = 2, 16
NW = NC * NS
IPW = N_I // NW
NBUF = 5
NROUNDS = N_J // NBUF - 1

_mesh = plsc.VectorSubcoreMesh(core_axis_name="c", subcore_axis_name="s")


@functools.partial(
    pl.kernel,
    mesh=_mesh,
    out_type=jax.ShapeDtypeStruct((N_J, N_I, DIM), jnp.float32),
    scratch_types=[
        pltpu.VMEM((N_J, IPW), jnp.int32),
        pltpu.VMEM((NBUF, IPW, DIM), jnp.float32),
        [pltpu.SemaphoreType.DMA for _ in range(NBUF)],
        [pltpu.SemaphoreType.DMA for _ in range(NBUF)],
    ],
)
def _embed_gather(idx_hbm, table_hbm, out_hbm, idx_v, rows_v, gsems, ssems):
    wid = lax.axis_index("s") * NC + lax.axis_index("c")
    ibase = wid * IPW
    pltpu.sync_copy(idx_hbm.at[:, pl.ds(ibase, IPW)], idx_v)

    def gather(j, b):
        return pltpu.make_async_copy(
            table_hbm.at[idx_v.at[j]], rows_v.at[b], gsems[b])

    def store(j, b):
        return pltpu.make_async_copy(
            rows_v.at[b], out_hbm.at[j, pl.ds(ibase, IPW)], ssems[b])

    for b in range(NBUF):
        gather(b, b).start()

    def body(g, carry):
        j0 = g * NBUF
        for b in range(NBUF):
            gather(j0 + b, b).wait()
            store(j0 + b, b).start()
        for b in range(NBUF):
            store(j0 + b, b).wait()
            gather(j0 + b + NBUF, b).start()
        return carry

    lax.fori_loop(0, NROUNDS, body, 0)

    j0 = NROUNDS * NBUF
    for b in range(NBUF):
        gather(j0 + b, b).wait()
        store(j0 + b, b).start()
    for b in range(NBUF):
        store(j0 + b, b).wait()


def kernel(x, weight):
    out_t = _embed_gather(x.T.astype(jnp.int32), weight)
    return out_t.transpose(1, 0, 2)


# speedup vs baseline: 3.1929x; 1.0303x over previous
import functools

import jax
import jax.numpy as jnp
from jax import lax
from jax.experimental import pallas as pl
from jax.experimental.pallas import tpu as pltpu
from jax.experimental.pallas import tpu_sc as plsc

N_I = 4096
N_J = 50
DIM = 128
NC, NS = 2, 16
NW = NC * NS
IPW = N_I // NW
HALF = IPW // 2
NCH = N_J * 2
NBUF = 10
NROUNDS = NCH // NBUF - 1

_mesh = plsc.VectorSubcoreMesh(core_axis_name="c", subcore_axis_name="s")


@functools.partial(
    pl.kernel,
    mesh=_mesh,
    out_type=jax.ShapeDtypeStruct((N_J, N_I, DIM), jnp.float32),
    scratch_types=[
        pltpu.VMEM((N_J, IPW), jnp.int32),
        pltpu.VMEM((NBUF, HALF, DIM), jnp.float32),
        [pltpu.SemaphoreType.DMA for _ in range(NBUF)],
        [pltpu.SemaphoreType.DMA for _ in range(NBUF)],
    ],
)
def _embed_gather(idx_hbm, table_hbm, out_hbm, idx_v, rows_v, gsems, ssems):
    wid = lax.axis_index("s") * NC + lax.axis_index("c")
    ibase = wid * IPW
    pltpu.sync_copy(idx_hbm.at[:, pl.ds(ibase, IPW)], idx_v)

    def gather(c, b):
        j, h = c // 2, c % 2
        return pltpu.make_async_copy(
            table_hbm.at[idx_v.at[j, pl.ds(h * HALF, HALF)]],
            rows_v.at[b], gsems[b])

    def store(c, b):
        j, h = c // 2, c % 2
        return pltpu.make_async_copy(
            rows_v.at[b], out_hbm.at[j, pl.ds(ibase + h * HALF, HALF)],
            ssems[b])

    for b in range(NBUF):
        gather(b, b).start()

    def body(g, carry):
        c0 = g * NBUF
        for b in range(NBUF):
            gather(c0 + b, b).wait()
            store(c0 + b, b).start()
        for b in range(NBUF):
            store(c0 + b, b).wait()
            gather(c0 + b + NBUF, b).start()
        return carry

    lax.fori_loop(0, NROUNDS, body, 0)

    c0 = NROUNDS * NBUF
    for b in range(NBUF):
        gather(c0 + b, b).wait()
        store(c0 + b, b).start()
    for b in range(NBUF):
        store(c0 + b, b).wait()


def kernel(x, weight):
    out_t = _embed_gather(x.T.astype(jnp.int32), weight)
    return out_t.transpose(1, 0, 2)
